# baseline (speedup 1.0000x reference)
import jax
import jax.numpy as jnp
from jax.experimental import pallas as pl
from jax.experimental.pallas import tpu as pltpu

_ROWS = 1_000_000
_DIM = 64
_STRIPES = 8
_STRIPE_ROWS = _ROWS // _STRIPES
_BLOCK_ROWS = 1_000
_GRID = _STRIPE_ROWS // _BLOCK_ROWS


def _l2norm_body(*refs):
    x_refs, o_refs = refs[:_STRIPES], refs[_STRIPES:]
    ones = jnp.ones((_DIM, _DIM), dtype=jnp.float32)
    for k in range(_STRIPES):
        x = x_refs[k][...]
        n = jax.lax.dot(x * x, ones, preferred_element_type=jnp.float32)
        o_refs[k][...] = x * jax.lax.rsqrt(n)


def kernel(weight):
    spec_in = [
        pl.BlockSpec((_BLOCK_ROWS, _DIM), lambda i, k=k: (k * _GRID + i, 0))
        for k in range(_STRIPES)
    ]
    spec_out = [
        pl.BlockSpec((_BLOCK_ROWS, _DIM), lambda i: (i, 0))
        for _ in range(_STRIPES)
    ]
    outs = pl.pallas_call(
        _l2norm_body,
        grid=(_GRID,),
        in_specs=spec_in,
        out_specs=spec_out,
        out_shape=[
            jax.ShapeDtypeStruct((_STRIPE_ROWS, _DIM), jnp.float32)
            for _ in range(_STRIPES)
        ],
        compiler_params=pltpu.CompilerParams(
            dimension_semantics=("arbitrary",),
        ),
    )(*([weight] * _STRIPES))
    return jnp.concatenate(outs, axis=0).reshape(_ROWS, _DIM)
